# baseline (speedup 1.0000x reference)
import functools

import numpy as np

import jax
import jax.numpy as jnp
from jax import lax
from jax.experimental import pallas as pl
from jax.experimental.pallas import tpu as pltpu
from jax.experimental.pallas import tpu_sc as plsc

BATCH = 64
NUM_PATCHES = 576
PATCH_DIM = 768
PROJ_DIM = 96
NUM_MASK = 432
NUM_UNMASK = 144

NW = 32
U_TOT = BATCH * NUM_UNMASK
M_TOT = BATCH * NUM_MASK
U_PER_W = U_TOT // NW
M_PER_W = M_TOT // NW
CHUNK = 96
POS_PAD = 128


def _threefry2x32(k1, k2, x0, x1):
    r0 = (13, 15, 26, 6)
    r1 = (17, 29, 16, 24)
    ks = (np.uint32(k1), np.uint32(k2),
          np.uint32(k1) ^ np.uint32(k2) ^ np.uint32(0x1BD11BDA))

    def rounds(x0, x1, rots):
        for r in rots:
            x0 = (x0 + x1).astype(np.uint32)
            x1 = (x1 << np.uint32(r)) | (x1 >> np.uint32(32 - r))
            x1 = x0 ^ x1
        return x0, x1

    with np.errstate(over="ignore"):
        x0 = (x0 + ks[0]).astype(np.uint32)
        x1 = (x1 + ks[1]).astype(np.uint32)
        for i, rots in enumerate((r0, r1, r0, r1, r0)):
            x0, x1 = rounds(x0, x1, rots)
            x0 = (x0 + ks[(i + 1) % 3]).astype(np.uint32)
            x1 = (x1 + ks[(i + 2) % 3] + np.uint32(i + 1)).astype(np.uint32)
    return x0, x1


def _masking_indices() -> np.ndarray:
    size = BATCH * NUM_PATCHES
    i64 = np.arange(size, dtype=np.uint64)
    c1 = (i64 >> np.uint64(32)).astype(np.uint32)
    c2 = (i64 & np.uint64(0xFFFFFFFF)).astype(np.uint32)
    b1, b2 = _threefry2x32(np.uint32(0), np.uint32(42), c1, c2)
    bits = (b1 ^ b2).reshape(BATCH, NUM_PATCHES)
    fb = (bits >> np.uint32(9)) | np.uint32(0x3F800000)
    u = np.maximum(np.float32(0), fb.view(np.float32) - np.float32(1.0))
    return np.argsort(u, axis=-1, kind="stable").astype(np.int32)


_RIDX = _masking_indices()
_MIDX = _RIDX[:, :NUM_MASK]
_UIDX = _RIDX[:, NUM_MASK:]
_MIDX_FLAT = np.ascontiguousarray(_MIDX.reshape(-1))
_UIDX_FLAT = np.ascontiguousarray(_UIDX.reshape(-1))
_UIDX_GLOB = np.ascontiguousarray(
    (_UIDX + np.arange(BATCH, dtype=np.int32)[:, None] * NUM_PATCHES)
    .reshape(-1))

def _sc_gather_body(patches_hbm, pos_plus_hbm, pos_hbm, uidxg_hbm, uidx_hbm,
                    midx_hbm, g_out, m_out, up_out,
                    uidxg_v, uidx_v, midx_v, prow_v, rrow_v, sem):
    wid = lax.axis_index("s") * 2 + lax.axis_index("c")
    ubase = wid * U_PER_W
    mbase = wid * M_PER_W
    pltpu.sync_copy(uidxg_hbm.at[pl.ds(ubase, U_PER_W)], uidxg_v)
    pltpu.sync_copy(uidx_hbm.at[pl.ds(ubase, U_PER_W)], uidx_v)
    pltpu.sync_copy(midx_hbm.at[pl.ds(mbase, M_PER_W)], midx_v)
    for c in range(U_PER_W // CHUNK):
        pltpu.async_copy(
            patches_hbm.at[uidxg_v.at[pl.ds(c * CHUNK, CHUNK)]], prow_v, sem
        ).wait()
        pltpu.sync_copy(prow_v, g_out.at[pl.ds(ubase + c * CHUNK, CHUNK)])
    for c in range(U_PER_W // CHUNK):
        pltpu.async_copy(
            pos_hbm.at[uidx_v.at[pl.ds(c * CHUNK, CHUNK)]], rrow_v, sem
        ).wait()
        pltpu.sync_copy(rrow_v, up_out.at[pl.ds(ubase + c * CHUNK, CHUNK)])
    for c in range(M_PER_W // CHUNK):
        pltpu.async_copy(
            pos_plus_hbm.at[midx_v.at[pl.ds(c * CHUNK, CHUNK)]], rrow_v, sem
        ).wait()
        pltpu.sync_copy(rrow_v, m_out.at[pl.ds(mbase + c * CHUNK, CHUNK)])


@functools.cache
def _sc_gather():
    mesh = plsc.VectorSubcoreMesh(core_axis_name="c", subcore_axis_name="s")
    return pl.kernel(
        _sc_gather_body,
        out_type=(
            jax.ShapeDtypeStruct((U_TOT, PATCH_DIM), jnp.float32),
            jax.ShapeDtypeStruct((M_TOT, POS_PAD), jnp.float32),
            jax.ShapeDtypeStruct((U_TOT, POS_PAD), jnp.float32),
        ),
        mesh=mesh,
        scratch_types=[
            pltpu.VMEM((U_PER_W,), jnp.int32),
            pltpu.VMEM((U_PER_W,), jnp.int32),
            pltpu.VMEM((M_PER_W,), jnp.int32),
            pltpu.VMEM((CHUNK, PATCH_DIM), jnp.float32),
            pltpu.VMEM((CHUNK, POS_PAD), jnp.float32),
            pltpu.SemaphoreType.DMA,
        ],
    )


def _pos_plus_body(mt_ref, w_ref, b_ref, pos_ref, out_ref):
    mtw = jnp.dot(mt_ref[...], w_ref[...]) + b_ref[...]
    out_ref[...] = pos_ref[...] + mtw


def _proj_body(x_ref, w_ref, b_ref, eye_ref, mpad_ref, uppad_ref,
               o_ref, mt_ref, up_ref):
    o_ref[...] = jnp.dot(x_ref[...], w_ref[...]) + b_ref[...]
    m_rows = mpad_ref[:, :PROJ_DIM]
    mt_ref[0] = lax.dot_general(
        m_rows, eye_ref[...], (((0,), (0,)), ((), ())),
        precision=lax.Precision.HIGHEST)
    up_ref[...] = uppad_ref[:, :PROJ_DIM]


def kernel(patches, W, b, pos_table, mask_token):
    mask_indices = jnp.asarray(_MIDX)
    unmask_indices = jnp.asarray(_UIDX)
    b2 = b.reshape(1, PROJ_DIM)
    w_pad = jnp.pad(W, ((0, 0), (0, POS_PAD - PROJ_DIM)))
    b_pad = jnp.pad(b2, ((0, 0), (0, POS_PAD - PROJ_DIM)))
    pos_pad = jnp.pad(pos_table, ((0, 0), (0, POS_PAD - PROJ_DIM)))

    pos_plus = pl.pallas_call(
        _pos_plus_body,
        out_shape=jax.ShapeDtypeStruct((NUM_PATCHES, POS_PAD), jnp.float32),
    )(mask_token, w_pad, b_pad, pos_pad)

    g_rows, m_pad_rows, up_pad_rows = _sc_gather()(
        patches.reshape(BATCH * NUM_PATCHES, PATCH_DIM),
        pos_plus, pos_pad,
        jnp.asarray(_UIDX_GLOB), jnp.asarray(_UIDX_FLAT),
        jnp.asarray(_MIDX_FLAT))

    eye = jnp.eye(NUM_MASK, dtype=jnp.float32)
    ue, mt, up_rows = pl.pallas_call(
        _proj_body,
        grid=(BATCH,),
        in_specs=[
            pl.BlockSpec((NUM_UNMASK, PATCH_DIM), lambda i: (i, 0)),
            pl.BlockSpec((PATCH_DIM, PROJ_DIM), lambda i: (0, 0)),
            pl.BlockSpec((1, PROJ_DIM), lambda i: (0, 0)),
            pl.BlockSpec((NUM_MASK, NUM_MASK), lambda i: (0, 0)),
            pl.BlockSpec((NUM_MASK, POS_PAD), lambda i: (i, 0)),
            pl.BlockSpec((NUM_UNMASK, POS_PAD), lambda i: (i, 0)),
        ],
        out_specs=[
            pl.BlockSpec((NUM_UNMASK, PROJ_DIM), lambda i: (i, 0)),
            pl.BlockSpec((1, PROJ_DIM, NUM_MASK), lambda i: (i, 0, 0)),
            pl.BlockSpec((NUM_UNMASK, PROJ_DIM), lambda i: (i, 0)),
        ],
        out_shape=[
            jax.ShapeDtypeStruct((U_TOT, PROJ_DIM), jnp.float32),
            jax.ShapeDtypeStruct((BATCH, PROJ_DIM, NUM_MASK), jnp.float32),
            jax.ShapeDtypeStruct((U_TOT, PROJ_DIM), jnp.float32),
        ],
    )(g_rows, W, b2, eye, m_pad_rows, up_pad_rows)

    return (
        ue.reshape(BATCH, NUM_UNMASK, PROJ_DIM),
        jnp.transpose(mt, (0, 2, 1)),
        up_rows.reshape(BATCH, NUM_UNMASK, PROJ_DIM),
        mask_indices,
        unmask_indices,
    )


# speedup vs baseline: 1.5013x; 1.5013x over previous
import functools

import numpy as np

import jax
import jax.numpy as jnp
from jax import lax
from jax.experimental import pallas as pl
from jax.experimental.pallas import tpu as pltpu
from jax.experimental.pallas import tpu_sc as plsc

BATCH = 64
NUM_PATCHES = 576
PATCH_DIM = 768
PROJ_DIM = 96
NUM_MASK = 432
NUM_UNMASK = 144

NW = 32
U_TOT = BATCH * NUM_UNMASK
U_PER_W = U_TOT // NW
CHUNK = 96


def _threefry2x32(k1, k2, x0, x1):
    r0 = (13, 15, 26, 6)
    r1 = (17, 29, 16, 24)
    ks = (np.uint32(k1), np.uint32(k2),
          np.uint32(k1) ^ np.uint32(k2) ^ np.uint32(0x1BD11BDA))

    def rounds(x0, x1, rots):
        for r in rots:
            x0 = (x0 + x1).astype(np.uint32)
            x1 = (x1 << np.uint32(r)) | (x1 >> np.uint32(32 - r))
            x1 = x0 ^ x1
        return x0, x1

    with np.errstate(over="ignore"):
        x0 = (x0 + ks[0]).astype(np.uint32)
        x1 = (x1 + ks[1]).astype(np.uint32)
        for i, rots in enumerate((r0, r1, r0, r1, r0)):
            x0, x1 = rounds(x0, x1, rots)
            x0 = (x0 + ks[(i + 1) % 3]).astype(np.uint32)
            x1 = (x1 + ks[(i + 2) % 3] + np.uint32(i + 1)).astype(np.uint32)
    return x0, x1


def _masking_indices() -> np.ndarray:
    size = BATCH * NUM_PATCHES
    i64 = np.arange(size, dtype=np.uint64)
    c1 = (i64 >> np.uint64(32)).astype(np.uint32)
    c2 = (i64 & np.uint64(0xFFFFFFFF)).astype(np.uint32)
    b1, b2 = _threefry2x32(np.uint32(0), np.uint32(42), c1, c2)
    bits = (b1 ^ b2).reshape(BATCH, NUM_PATCHES)
    fb = (bits >> np.uint32(9)) | np.uint32(0x3F800000)
    u = np.maximum(np.float32(0), fb.view(np.float32) - np.float32(1.0))
    return np.argsort(u, axis=-1, kind="stable").astype(np.int32)


_RIDX = _masking_indices()
_MIDX = _RIDX[:, :NUM_MASK]
_UIDX = _RIDX[:, NUM_MASK:]
_UIDX_GLOB = np.ascontiguousarray(
    (_UIDX + np.arange(BATCH, dtype=np.int32)[:, None] * NUM_PATCHES)
    .reshape(-1))


def _sc_gather_body(patches_hbm, uidxg_hbm, g_out, uidxg_v, prow_v, sem):
    wid = lax.axis_index("s") * 2 + lax.axis_index("c")
    ubase = wid * U_PER_W
    pltpu.sync_copy(uidxg_hbm.at[pl.ds(ubase, U_PER_W)], uidxg_v)
    for c in range(U_PER_W // CHUNK):
        pltpu.async_copy(
            patches_hbm.at[uidxg_v.at[pl.ds(c * CHUNK, CHUNK)]], prow_v, sem
        ).wait()
        pltpu.sync_copy(prow_v, g_out.at[pl.ds(ubase + c * CHUNK, CHUNK)])


@functools.cache
def _sc_gather():
    mesh = plsc.VectorSubcoreMesh(core_axis_name="c", subcore_axis_name="s")
    return pl.kernel(
        _sc_gather_body,
        out_type=jax.ShapeDtypeStruct((U_TOT, PATCH_DIM), jnp.float32),
        mesh=mesh,
        scratch_types=[
            pltpu.VMEM((U_PER_W,), jnp.int32),
            pltpu.VMEM((CHUNK, PATCH_DIM), jnp.float32),
            pltpu.SemaphoreType.DMA,
        ],
    )


def _pos_plus_body(mt_ref, w_ref, b_ref, pos_ref, out_ref):
    mtw = jnp.dot(mt_ref[...], w_ref[...]) + b_ref[...]
    out_ref[...] = pos_ref[...] + mtw


def _proj_body(x_ref, w_ref, b_ref, pp_ref, pos_ref, midx_ref, uidx_ref,
               o_ref, mt_ref, up_ref):
    o_ref[...] = jnp.dot(x_ref[...], w_ref[...]) + b_ref[...]
    iota_m = lax.broadcasted_iota(jnp.int32, (NUM_PATCHES, NUM_MASK), 0)
    oh_m = (iota_m == midx_ref[0]).astype(jnp.float32)
    mt_ref[0] = lax.dot_general(
        pp_ref[...], oh_m, (((0,), (0,)), ((), ())))
    iota_u = lax.broadcasted_iota(jnp.int32, (NUM_UNMASK, NUM_PATCHES), 1)
    oh_u = (iota_u == uidx_ref[0].reshape(NUM_UNMASK, 1)).astype(jnp.float32)
    up_ref[0] = jnp.dot(oh_u, pos_ref[...])


def kernel(patches, W, b, pos_table, mask_token):
    mask_indices = jnp.asarray(_MIDX)
    unmask_indices = jnp.asarray(_UIDX)
    b2 = b.reshape(1, PROJ_DIM)

    pos_plus = pl.pallas_call(
        _pos_plus_body,
        out_shape=jax.ShapeDtypeStruct((NUM_PATCHES, PROJ_DIM), jnp.float32),
    )(mask_token, W, b2, pos_table)

    g_rows = _sc_gather()(
        patches.reshape(BATCH * NUM_PATCHES, PATCH_DIM),
        jnp.asarray(_UIDX_GLOB))

    ue, mt, up = pl.pallas_call(
        _proj_body,
        grid=(BATCH,),
        in_specs=[
            pl.BlockSpec((NUM_UNMASK, PATCH_DIM), lambda i: (i, 0)),
            pl.BlockSpec((PATCH_DIM, PROJ_DIM), lambda i: (0, 0)),
            pl.BlockSpec((1, PROJ_DIM), lambda i: (0, 0)),
            pl.BlockSpec((NUM_PATCHES, PROJ_DIM), lambda i: (0, 0)),
            pl.BlockSpec((NUM_PATCHES, PROJ_DIM), lambda i: (0, 0)),
            pl.BlockSpec((1, 1, NUM_MASK), lambda i: (i, 0, 0)),
            pl.BlockSpec((1, 1, NUM_UNMASK), lambda i: (i, 0, 0)),
        ],
        out_specs=[
            pl.BlockSpec((NUM_UNMASK, PROJ_DIM), lambda i: (i, 0)),
            pl.BlockSpec((1, PROJ_DIM, NUM_MASK), lambda i: (i, 0, 0)),
            pl.BlockSpec((1, NUM_UNMASK, PROJ_DIM), lambda i: (i, 0, 0)),
        ],
        out_shape=[
            jax.ShapeDtypeStruct((U_TOT, PROJ_DIM), jnp.float32),
            jax.ShapeDtypeStruct((BATCH, PROJ_DIM, NUM_MASK), jnp.float32),
            jax.ShapeDtypeStruct((BATCH, NUM_UNMASK, PROJ_DIM), jnp.float32),
        ],
    )(g_rows, W, b2, pos_plus, pos_table,
      jnp.asarray(_MIDX).reshape(BATCH, 1, NUM_MASK),
      jnp.asarray(_UIDX).reshape(BATCH, 1, NUM_UNMASK))

    return (
        ue.reshape(BATCH, NUM_UNMASK, PROJ_DIM),
        jnp.transpose(mt, (0, 2, 1)),
        up,
        mask_indices,
        unmask_indices,
    )


# speedup vs baseline: 2.0843x; 1.3883x over previous
import functools

import numpy as np

import jax
import jax.numpy as jnp
from jax import lax
from jax.experimental import pallas as pl
from jax.experimental.pallas import tpu as pltpu
from jax.experimental.pallas import tpu_sc as plsc

BATCH = 64
NUM_PATCHES = 576
PATCH_DIM = 768
PROJ_DIM = 96
NUM_MASK = 432
NUM_UNMASK = 144

NW = 32
U_TOT = BATCH * NUM_UNMASK
U_PER_W = U_TOT // NW
CHUNK = 96


def _threefry2x32(k1, k2, x0, x1):
    r0 = (13, 15, 26, 6)
    r1 = (17, 29, 16, 24)
    ks = (np.uint32(k1), np.uint32(k2),
          np.uint32(k1) ^ np.uint32(k2) ^ np.uint32(0x1BD11BDA))

    def rounds(x0, x1, rots):
        for r in rots:
            x0 = (x0 + x1).astype(np.uint32)
            x1 = (x1 << np.uint32(r)) | (x1 >> np.uint32(32 - r))
            x1 = x0 ^ x1
        return x0, x1

    with np.errstate(over="ignore"):
        x0 = (x0 + ks[0]).astype(np.uint32)
        x1 = (x1 + ks[1]).astype(np.uint32)
        for i, rots in enumerate((r0, r1, r0, r1, r0)):
            x0, x1 = rounds(x0, x1, rots)
            x0 = (x0 + ks[(i + 1) % 3]).astype(np.uint32)
            x1 = (x1 + ks[(i + 2) % 3] + np.uint32(i + 1)).astype(np.uint32)
    return x0, x1


def _masking_indices() -> np.ndarray:
    size = BATCH * NUM_PATCHES
    i64 = np.arange(size, dtype=np.uint64)
    c1 = (i64 >> np.uint64(32)).astype(np.uint32)
    c2 = (i64 & np.uint64(0xFFFFFFFF)).astype(np.uint32)
    b1, b2 = _threefry2x32(np.uint32(0), np.uint32(42), c1, c2)
    bits = (b1 ^ b2).reshape(BATCH, NUM_PATCHES)
    fb = (bits >> np.uint32(9)) | np.uint32(0x3F800000)
    u = np.maximum(np.float32(0), fb.view(np.float32) - np.float32(1.0))
    return np.argsort(u, axis=-1, kind="stable").astype(np.int32)


_RIDX = _masking_indices()
_MIDX = _RIDX[:, :NUM_MASK]
_UIDX = _RIDX[:, NUM_MASK:]
_UIDX_GLOB = np.ascontiguousarray(
    (_UIDX + np.arange(BATCH, dtype=np.int32)[:, None] * NUM_PATCHES)
    .reshape(-1))


def _sc_gather_body(patches_hbm, uidxg_hbm, g_out, uidxg_v, prow_v, sem):
    wid = lax.axis_index("s") * 2 + lax.axis_index("c")
    ubase = wid * U_PER_W
    pltpu.sync_copy(uidxg_hbm.at[pl.ds(ubase, U_PER_W)], uidxg_v)
    for c in range(U_PER_W // CHUNK):
        pltpu.async_copy(
            patches_hbm.at[uidxg_v.at[pl.ds(c * CHUNK, CHUNK)]], prow_v, sem
        ).wait()
        pltpu.sync_copy(prow_v, g_out.at[pl.ds(ubase + c * CHUNK, CHUNK)])


@functools.cache
def _sc_gather():
    mesh = plsc.VectorSubcoreMesh(core_axis_name="c", subcore_axis_name="s")
    return pl.kernel(
        _sc_gather_body,
        out_type=jax.ShapeDtypeStruct((U_TOT, PATCH_DIM), jnp.float32),
        mesh=mesh,
        scratch_types=[
            pltpu.VMEM((U_PER_W,), jnp.int32),
            pltpu.VMEM((CHUNK, PATCH_DIM), jnp.float32),
            pltpu.SemaphoreType.DMA,
        ],
    )


def _pos_plus_t_body(mt_ref, w_ref, bt_ref, post_ref, out_ref):
    mtwt = lax.dot_general(
        w_ref[...], mt_ref[...], (((0,), (1,)), ((), ())))
    out_ref[...] = post_ref[...] + (mtwt + bt_ref[...])


_RB = 4
_NSTEP = BATCH // _RB


def _proj_body(x_ref, w_ref, b_ref, ppt_ref, pos_ref, midx_ref, uidx_ref,
               o_ref, mt_ref, up_ref):
    o_ref[...] = jnp.dot(x_ref[...], w_ref[...]) + b_ref[...]
    iota_m = lax.broadcasted_iota(jnp.int32, (NUM_PATCHES, NUM_MASK), 0)
    iota_u = lax.broadcasted_iota(jnp.int32, (NUM_UNMASK, NUM_PATCHES), 1)
    for r in range(_RB):
        oh_m = (iota_m == midx_ref[r]).astype(jnp.float32)
        mt_ref[r] = jnp.dot(ppt_ref[...], oh_m)
        oh_u = (iota_u == uidx_ref[r].reshape(NUM_UNMASK, 1)).astype(
            jnp.float32)
        up_ref[r] = jnp.dot(oh_u, pos_ref[...])


def kernel(patches, W, b, pos_table, mask_token):
    mask_indices = jnp.asarray(_MIDX)
    unmask_indices = jnp.asarray(_UIDX)
    b2 = b.reshape(1, PROJ_DIM)

    pos_plus_t = pl.pallas_call(
        _pos_plus_t_body,
        out_shape=jax.ShapeDtypeStruct((PROJ_DIM, NUM_PATCHES), jnp.float32),
    )(mask_token, W, b.reshape(PROJ_DIM, 1), pos_table.T)

    g_rows = _sc_gather()(
        patches.reshape(BATCH * NUM_PATCHES, PATCH_DIM),
        jnp.asarray(_UIDX_GLOB))

    ue, mt, up = pl.pallas_call(
        _proj_body,
        grid=(_NSTEP,),
        in_specs=[
            pl.BlockSpec((_RB * NUM_UNMASK, PATCH_DIM), lambda i: (i, 0)),
            pl.BlockSpec((PATCH_DIM, PROJ_DIM), lambda i: (0, 0)),
            pl.BlockSpec((1, PROJ_DIM), lambda i: (0, 0)),
            pl.BlockSpec((PROJ_DIM, NUM_PATCHES), lambda i: (0, 0)),
            pl.BlockSpec((NUM_PATCHES, PROJ_DIM), lambda i: (0, 0)),
            pl.BlockSpec((_RB, 1, NUM_MASK), lambda i: (i, 0, 0)),
            pl.BlockSpec((_RB, 1, NUM_UNMASK), lambda i: (i, 0, 0)),
        ],
        out_specs=[
            pl.BlockSpec((_RB * NUM_UNMASK, PROJ_DIM), lambda i: (i, 0)),
            pl.BlockSpec((_RB, PROJ_DIM, NUM_MASK), lambda i: (i, 0, 0)),
            pl.BlockSpec((_RB, NUM_UNMASK, PROJ_DIM), lambda i: (i, 0, 0)),
        ],
        out_shape=[
            jax.ShapeDtypeStruct((U_TOT, PROJ_DIM), jnp.float32),
            jax.ShapeDtypeStruct((BATCH, PROJ_DIM, NUM_MASK), jnp.float32),
            jax.ShapeDtypeStruct((BATCH, NUM_UNMASK, PROJ_DIM), jnp.float32),
        ],
    )(g_rows, W, b2, pos_plus_t, pos_table,
      jnp.asarray(_MIDX).reshape(BATCH, 1, NUM_MASK),
      jnp.asarray(_UIDX).reshape(BATCH, 1, NUM_UNMASK))

    return (
        ue.reshape(BATCH, NUM_UNMASK, PROJ_DIM),
        jnp.transpose(mt, (0, 2, 1)),
        up,
        mask_indices,
        unmask_indices,
    )


# speedup vs baseline: 2.3046x; 1.1057x over previous
import functools

import numpy as np

import jax
import jax.numpy as jnp
from jax import lax
from jax.experimental import pallas as pl
from jax.experimental.pallas import tpu as pltpu
from jax.experimental.pallas import tpu_sc as plsc

BATCH = 64
NUM_PATCHES = 576
PATCH_DIM = 768
PROJ_DIM = 96
NUM_MASK = 432
NUM_UNMASK = 144

NW = 32
U_TOT = BATCH * NUM_UNMASK
U_PER_W = U_TOT // NW
CHUNK = 96


def _threefry2x32(k1, k2, x0, x1):
    r0 = (13, 15, 26, 6)
    r1 = (17, 29, 16, 24)
    ks = (np.uint32(k1), np.uint32(k2),
          np.uint32(k1) ^ np.uint32(k2) ^ np.uint32(0x1BD11BDA))

    def rounds(x0, x1, rots):
        for r in rots:
            x0 = (x0 + x1).astype(np.uint32)
            x1 = (x1 << np.uint32(r)) | (x1 >> np.uint32(32 - r))
            x1 = x0 ^ x1
        return x0, x1

    with np.errstate(over="ignore"):
        x0 = (x0 + ks[0]).astype(np.uint32)
        x1 = (x1 + ks[1]).astype(np.uint32)
        for i, rots in enumerate((r0, r1, r0, r1, r0)):
            x0, x1 = rounds(x0, x1, rots)
            x0 = (x0 + ks[(i + 1) % 3]).astype(np.uint32)
            x1 = (x1 + ks[(i + 2) % 3] + np.uint32(i + 1)).astype(np.uint32)
    return x0, x1


def _masking_indices() -> np.ndarray:
    size = BATCH * NUM_PATCHES
    i64 = np.arange(size, dtype=np.uint64)
    c1 = (i64 >> np.uint64(32)).astype(np.uint32)
    c2 = (i64 & np.uint64(0xFFFFFFFF)).astype(np.uint32)
    b1, b2 = _threefry2x32(np.uint32(0), np.uint32(42), c1, c2)
    bits = (b1 ^ b2).reshape(BATCH, NUM_PATCHES)
    fb = (bits >> np.uint32(9)) | np.uint32(0x3F800000)
    u = np.maximum(np.float32(0), fb.view(np.float32) - np.float32(1.0))
    return np.argsort(u, axis=-1, kind="stable").astype(np.int32)


_RIDX = _masking_indices()
_MIDX = _RIDX[:, :NUM_MASK]
_UIDX = _RIDX[:, NUM_MASK:]
_UIDX_GLOB = np.ascontiguousarray(
    (_UIDX + np.arange(BATCH, dtype=np.int32)[:, None] * NUM_PATCHES)
    .reshape(-1))


def _sc_gather_body(patches_hbm, uidxg_hbm, g_out, uidxg_v, prow_v, sem):
    wid = lax.axis_index("s") * 2 + lax.axis_index("c")
    ubase = wid * U_PER_W
    pltpu.sync_copy(uidxg_hbm.at[pl.ds(ubase, U_PER_W)], uidxg_v)
    for c in range(U_PER_W // CHUNK):
        pltpu.async_copy(
            patches_hbm.at[uidxg_v.at[pl.ds(c * CHUNK, CHUNK)]], prow_v, sem
        ).wait()
        pltpu.sync_copy(prow_v, g_out.at[pl.ds(ubase + c * CHUNK, CHUNK)])


@functools.cache
def _sc_gather():
    mesh = plsc.VectorSubcoreMesh(core_axis_name="c", subcore_axis_name="s")
    return pl.kernel(
        _sc_gather_body,
        out_type=jax.ShapeDtypeStruct((U_TOT, PATCH_DIM), jnp.float32),
        mesh=mesh,
        scratch_types=[
            pltpu.VMEM((U_PER_W,), jnp.int32),
            pltpu.VMEM((CHUNK, PATCH_DIM), jnp.float32),
            pltpu.SemaphoreType.DMA,
        ],
    )


def _pos_plus_t_body(mt_ref, w_ref, bt_ref, post_ref, out_ref):
    mtwt = lax.dot_general(
        w_ref[...], mt_ref[...], (((0,), (1,)), ((), ())))
    out_ref[...] = post_ref[...] + (mtwt + bt_ref[...])


_RB = 4
_NSTEP = BATCH // _RB
_PROJ_BLK = 1024


def _onehot_body(ppt_ref, pos_ref, midx_ref, uidx_ref, mt_ref, up_ref):
    iota_m = lax.broadcasted_iota(jnp.int32, (NUM_PATCHES, NUM_MASK), 0)
    iota_u = lax.broadcasted_iota(jnp.int32, (NUM_UNMASK, NUM_PATCHES), 1)
    for r in range(_RB):
        oh_m = (iota_m == midx_ref[r]).astype(jnp.float32)
        mt_ref[r] = jnp.dot(ppt_ref[...], oh_m)
        oh_u = (iota_u == uidx_ref[r].reshape(NUM_UNMASK, 1)).astype(
            jnp.float32)
        up_ref[r] = jnp.dot(oh_u, pos_ref[...])


def _proj_body(x_ref, w_ref, b_ref, o_ref):
    o_ref[...] = jnp.dot(x_ref[...], w_ref[...]) + b_ref[...]


def kernel(patches, W, b, pos_table, mask_token):
    mask_indices = jnp.asarray(_MIDX)
    unmask_indices = jnp.asarray(_UIDX)
    b2 = b.reshape(1, PROJ_DIM)

    pos_plus_t = pl.pallas_call(
        _pos_plus_t_body,
        out_shape=jax.ShapeDtypeStruct((PROJ_DIM, NUM_PATCHES), jnp.float32),
    )(mask_token, W, b.reshape(PROJ_DIM, 1), pos_table.T)

    g_rows = _sc_gather()(
        patches.reshape(BATCH * NUM_PATCHES, PATCH_DIM),
        jnp.asarray(_UIDX_GLOB))

    mt, up = pl.pallas_call(
        _onehot_body,
        grid=(_NSTEP,),
        in_specs=[
            pl.BlockSpec((PROJ_DIM, NUM_PATCHES), lambda i: (0, 0)),
            pl.BlockSpec((NUM_PATCHES, PROJ_DIM), lambda i: (0, 0)),
            pl.BlockSpec((_RB, 1, NUM_MASK), lambda i: (i, 0, 0)),
            pl.BlockSpec((_RB, 1, NUM_UNMASK), lambda i: (i, 0, 0)),
        ],
        out_specs=[
            pl.BlockSpec((_RB, PROJ_DIM, NUM_MASK), lambda i: (i, 0, 0)),
            pl.BlockSpec((_RB, NUM_UNMASK, PROJ_DIM), lambda i: (i, 0, 0)),
        ],
        out_shape=[
            jax.ShapeDtypeStruct((BATCH, PROJ_DIM, NUM_MASK), jnp.float32),
            jax.ShapeDtypeStruct((BATCH, NUM_UNMASK, PROJ_DIM), jnp.float32),
        ],
    )(pos_plus_t, pos_table,
      jnp.asarray(_MIDX).reshape(BATCH, 1, NUM_MASK),
      jnp.asarray(_UIDX).reshape(BATCH, 1, NUM_UNMASK))

    ue = pl.pallas_call(
        _proj_body,
        grid=(U_TOT // _PROJ_BLK,),
        in_specs=[
            pl.BlockSpec((_PROJ_BLK, PATCH_DIM), lambda i: (i, 0)),
            pl.BlockSpec((PATCH_DIM, PROJ_DIM), lambda i: (0, 0)),
            pl.BlockSpec((1, PROJ_DIM), lambda i: (0, 0)),
        ],
        out_specs=pl.BlockSpec((_PROJ_BLK, PROJ_DIM), lambda i: (i, 0)),
        out_shape=jax.ShapeDtypeStruct((U_TOT, PROJ_DIM), jnp.float32),
    )(g_rows, W, b2)

    return (
        ue.reshape(BATCH, NUM_UNMASK, PROJ_DIM),
        jnp.transpose(mt, (0, 2, 1)),
        up,
        mask_indices,
        unmask_indices,
    )
